# baseline (speedup 1.0000x reference)
import functools

import jax
import jax.numpy as jnp
from jax import lax
from jax.experimental import pallas as pl
from jax.experimental.pallas import tpu as pltpu
from jax.experimental.pallas import tpu_sc as plsc

BATCH = 16384
LATENT = 128
HALF = BATCH // 2


def _sc_gather_chunk(Eu, Ei, user, items, off, count):
    info = plsc.get_sparse_core_info()
    NC, NS = info.num_cores, info.num_subcores
    NW = NC * NS
    bpw = count // NW

    mesh = plsc.VectorSubcoreMesh(core_axis_name="c", subcore_axis_name="s")

    @functools.partial(
        pl.kernel,
        mesh=mesh,
        out_type=(
            jax.ShapeDtypeStruct((count, LATENT), jnp.float32),
            jax.ShapeDtypeStruct((count, LATENT), jnp.float32),
        ),
        scratch_types=[
            pltpu.VMEM((bpw,), jnp.int32),
            pltpu.VMEM((bpw, LATENT), jnp.float32),
            pltpu.SemaphoreType.DMA,
        ],
    )
    def k(eu_hbm, ei_hbm, u_hbm, it_hbm, outu_hbm, outi_hbm, idx_v, rows_v, sem):
        wid = lax.axis_index("s") * NC + lax.axis_index("c")
        base = wid * bpw
        pltpu.sync_copy(u_hbm.at[pl.ds(off + base, bpw)], idx_v)
        pltpu.async_copy(eu_hbm.at[idx_v], rows_v, sem).wait()
        pltpu.sync_copy(rows_v, outu_hbm.at[pl.ds(base, bpw)])
        pltpu.sync_copy(it_hbm.at[pl.ds(off + base, bpw)], idx_v)
        pltpu.async_copy(ei_hbm.at[idx_v], rows_v, sem).wait()
        pltpu.sync_copy(rows_v, outi_hbm.at[pl.ds(base, bpw)])

    return k(Eu, Ei, user, items)


def _mlp_body(ue_ref, ie_ref, w1u_ref, w1i_ref, b1_ref, w2_ref, b2_ref,
              w3_ref, b3_ref, w4_ref, b4_ref, out_ref):
    x = jnp.dot(ue_ref[...].astype(jnp.bfloat16), w1u_ref[...],
                preferred_element_type=jnp.float32)
    x = x + jnp.dot(ie_ref[...].astype(jnp.bfloat16), w1i_ref[...],
                    preferred_element_type=jnp.float32)
    x = jnp.maximum(x + b1_ref[...], 0.0)
    x = jnp.maximum(jnp.dot(x, w2_ref[...], preferred_element_type=jnp.float32) + b2_ref[...], 0.0)
    x = jnp.maximum(jnp.dot(x, w3_ref[...], preferred_element_type=jnp.float32) + b3_ref[...], 0.0)
    x = jnp.dot(x, w4_ref[...], preferred_element_type=jnp.float32) + b4_ref[...]
    out_ref[...] = (1.0 / (1.0 + jnp.exp2(x))).astype(jnp.bfloat16)


def _tc_mlp(ue, ie, w1u, w1i, b1r, w2t, b2r, w3t, b3r, w4t, b4r):
    batch = ue.shape[0]
    BLK = 4096
    grid = (batch // BLK,)

    def full2(shape):
        return pl.BlockSpec(shape, lambda i: (0, 0))

    return pl.pallas_call(
        _mlp_body,
        grid=grid,
        in_specs=[
            pl.BlockSpec((BLK, LATENT), lambda i: (i, 0)),
            pl.BlockSpec((BLK, LATENT), lambda i: (i, 0)),
            full2(w1u.shape), full2(w1i.shape), full2(b1r.shape),
            full2(w2t.shape), full2(b2r.shape),
            full2(w3t.shape), full2(b3r.shape),
            full2(w4t.shape), full2(b4r.shape),
        ],
        out_specs=pl.BlockSpec((BLK, 1), lambda i: (i, 0)),
        out_shape=jax.ShapeDtypeStruct((batch, 1), jnp.bfloat16),
    )(ue, ie, w1u, w1i, b1r, w2t, b2r, w3t, b3r, w4t, b4r)


def kernel(user, items, Eu, Ei, W1, b1, W2, b2, W3, b3, W4, b4):
    u32 = user.astype(jnp.int32)
    i32 = items.astype(jnp.int32)
    w1u = W1[:, :LATENT].T.astype(jnp.bfloat16)
    w1i = W1[:, LATENT:].T.astype(jnp.bfloat16)
    w2t = W2.T
    w3t = W3.T
    NEG_LOG2E = -1.4426950408889634
    w4t = W4.T * NEG_LOG2E
    b1r = b1.reshape(1, -1)
    b2r = b2.reshape(1, -1)
    b3r = b3.reshape(1, -1)
    b4r = b4.reshape(1, -1) * NEG_LOG2E
    SPLIT = BATCH // 2
    ue0, ie0 = _sc_gather_chunk(Eu, Ei, u32, i32, 0, SPLIT)
    ue1, ie1 = _sc_gather_chunk(Eu, Ei, u32, i32, SPLIT, BATCH - SPLIT)
    y0 = _tc_mlp(ue0, ie0, w1u, w1i, b1r, w2t, b2r, w3t, b3r, w4t, b4r)
    y1 = _tc_mlp(ue1, ie1, w1u, w1i, b1r, w2t, b2r, w3t, b3r, w4t, b4r)
    return jnp.concatenate([y0, y1], axis=0).astype(jnp.float32)


# speedup vs baseline: 1.0527x; 1.0527x over previous
import functools

import jax
import jax.numpy as jnp
from jax import lax
from jax.experimental import pallas as pl
from jax.experimental.pallas import tpu as pltpu
from jax.experimental.pallas import tpu_sc as plsc

BATCH = 16384
LATENT = 128
HALF = BATCH // 2


def _sc_gather_chunk(Eu, Ei, user, items, off, count):
    info = plsc.get_sparse_core_info()
    NC, NS = info.num_cores, info.num_subcores
    NW = NC * NS
    bpw = count // NW

    mesh = plsc.VectorSubcoreMesh(core_axis_name="c", subcore_axis_name="s")

    @functools.partial(
        pl.kernel,
        mesh=mesh,
        out_type=(
            jax.ShapeDtypeStruct((count, LATENT), jnp.float32),
            jax.ShapeDtypeStruct((count, LATENT), jnp.float32),
        ),
        scratch_types=[
            pltpu.VMEM((bpw,), jnp.int32),
            pltpu.VMEM((bpw,), jnp.int32),
            pltpu.VMEM((bpw, LATENT), jnp.float32),
            pltpu.VMEM((bpw, LATENT), jnp.float32),
            pltpu.SemaphoreType.DMA,
            pltpu.SemaphoreType.DMA,
            pltpu.SemaphoreType.DMA,
            pltpu.SemaphoreType.DMA,
        ],
    )
    def k(eu_hbm, ei_hbm, u_hbm, it_hbm, outu_hbm, outi_hbm,
          idxu_v, idxi_v, rows_u, rows_i, isem_u, isem_i, wsem_u, wsem_i):
        wid = lax.axis_index("s") * NC + lax.axis_index("c")
        base = wid * bpw
        liu = pltpu.async_copy(u_hbm.at[pl.ds(off + base, bpw)], idxu_v, isem_u)
        lii = pltpu.async_copy(it_hbm.at[pl.ds(off + base, bpw)], idxi_v, isem_i)
        liu.wait()
        gu = pltpu.async_copy(eu_hbm.at[idxu_v], rows_u, isem_u)
        lii.wait()
        gi = pltpu.async_copy(ei_hbm.at[idxi_v], rows_i, isem_i)
        gu.wait()
        wu = pltpu.async_copy(rows_u, outu_hbm.at[pl.ds(base, bpw)], wsem_u)
        gi.wait()
        wi = pltpu.async_copy(rows_i, outi_hbm.at[pl.ds(base, bpw)], wsem_i)
        wu.wait()
        wi.wait()

    return k(Eu, Ei, user, items)


def _mlp_body(ue_ref, ie_ref, w1u_ref, w1i_ref, b1_ref, w2_ref, b2_ref,
              w3_ref, b3_ref, w4_ref, b4_ref, out_ref):
    x = jnp.dot(ue_ref[...].astype(jnp.bfloat16), w1u_ref[...],
                preferred_element_type=jnp.float32)
    x = x + jnp.dot(ie_ref[...].astype(jnp.bfloat16), w1i_ref[...],
                    preferred_element_type=jnp.float32)
    x = jnp.maximum(x + b1_ref[...], 0.0)
    x = jnp.maximum(jnp.dot(x, w2_ref[...], preferred_element_type=jnp.float32) + b2_ref[...], 0.0)
    x = jnp.maximum(jnp.dot(x, w3_ref[...], preferred_element_type=jnp.float32) + b3_ref[...], 0.0)
    x = jnp.dot(x, w4_ref[...], preferred_element_type=jnp.float32) + b4_ref[...]
    out_ref[...] = (1.0 / (1.0 + jnp.exp2(x))).astype(jnp.bfloat16)


def _tc_mlp(ue, ie, w1u, w1i, b1r, w2t, b2r, w3t, b3r, w4t, b4r):
    batch = ue.shape[0]
    BLK = batch // 2
    grid = (batch // BLK,)

    def full2(shape):
        return pl.BlockSpec(shape, lambda i: (0, 0))

    return pl.pallas_call(
        _mlp_body,
        grid=grid,
        in_specs=[
            pl.BlockSpec((BLK, LATENT), lambda i: (i, 0)),
            pl.BlockSpec((BLK, LATENT), lambda i: (i, 0)),
            full2(w1u.shape), full2(w1i.shape), full2(b1r.shape),
            full2(w2t.shape), full2(b2r.shape),
            full2(w3t.shape), full2(b3r.shape),
            full2(w4t.shape), full2(b4r.shape),
        ],
        out_specs=pl.BlockSpec((BLK, 1), lambda i: (i, 0)),
        out_shape=jax.ShapeDtypeStruct((batch, 1), jnp.bfloat16),
    )(ue, ie, w1u, w1i, b1r, w2t, b2r, w3t, b3r, w4t, b4r)


def kernel(user, items, Eu, Ei, W1, b1, W2, b2, W3, b3, W4, b4):
    u32 = user.astype(jnp.int32)
    i32 = items.astype(jnp.int32)
    w1u = W1[:, :LATENT].T.astype(jnp.bfloat16)
    w1i = W1[:, LATENT:].T.astype(jnp.bfloat16)
    w2t = W2.T
    w3t = W3.T
    NEG_LOG2E = -1.4426950408889634
    w4t = W4.T * NEG_LOG2E
    b1r = b1.reshape(1, -1)
    b2r = b2.reshape(1, -1)
    b3r = b3.reshape(1, -1)
    b4r = b4.reshape(1, -1) * NEG_LOG2E
    SPLIT = 9216
    ue0, ie0 = _sc_gather_chunk(Eu, Ei, u32, i32, 0, SPLIT)
    ue1, ie1 = _sc_gather_chunk(Eu, Ei, u32, i32, SPLIT, BATCH - SPLIT)
    y0 = _tc_mlp(ue0, ie0, w1u, w1i, b1r, w2t, b2r, w3t, b3r, w4t, b4r)
    y1 = _tc_mlp(ue1, ie1, w1u, w1i, b1r, w2t, b2r, w3t, b3r, w4t, b4r)
    return jnp.concatenate([y0, y1], axis=0).astype(jnp.float32)
